# Initial kernel scaffold; baseline (speedup 1.0000x reference)
import jax
import jax.numpy as jnp
from jax.experimental import pallas as pl


def kernel(data, adj, W, b):
    raise NotImplementedError("write your pallas kernel here")


# speedup vs baseline: 1.6172x; 1.6172x over previous
import jax
import jax.numpy as jnp
from jax.experimental import pallas as pl
from jax.experimental.pallas import tpu as pltpu


def _cheb_kernel(data_ref, adj_ref, w_ref, b_ref, out_ref):
    adj = adj_ref[...]
    deg = jnp.sum(adj, axis=1, keepdims=True)
    dinv = jnp.where(deg > 0, deg ** -0.5, 0.0)

    x0 = data_ref[0]

    def mop(v):
        sv = dinv * v
        u = jax.lax.dot_general(
            adj, sv, (((0,), (0,)), ((), ())),
            preferred_element_type=jnp.float32)
        return -dinv * u

    x1 = mop(x0)
    x2 = 2.0 * mop(x1) - x0

    acc = jnp.dot(x0, w_ref[0], preferred_element_type=jnp.float32)
    acc = acc + jnp.dot(x1, w_ref[1], preferred_element_type=jnp.float32)
    acc = acc + jnp.dot(x2, w_ref[2], preferred_element_type=jnp.float32)
    out_ref[0] = acc + b_ref[...]


def kernel(data, adj, W, b):
    B, N, F_IN = data.shape
    K, _, F_OUT = W.shape
    b2 = b.reshape(1, F_OUT)
    return pl.pallas_call(
        _cheb_kernel,
        grid=(B,),
        in_specs=[
            pl.BlockSpec((1, N, F_IN), lambda i: (i, 0, 0)),
            pl.BlockSpec((N, N), lambda i: (0, 0)),
            pl.BlockSpec((K, F_IN, F_OUT), lambda i: (0, 0, 0)),
            pl.BlockSpec((1, F_OUT), lambda i: (0, 0)),
        ],
        out_specs=pl.BlockSpec((1, N, F_OUT), lambda i: (i, 0, 0)),
        out_shape=jax.ShapeDtypeStruct((B, N, F_OUT), jnp.float32),
        compiler_params=pltpu.CompilerParams(
            dimension_semantics=("parallel",),
        ),
    )(data, adj, W, b2)
